# Initial kernel scaffold; baseline (speedup 1.0000x reference)
import jax
import jax.numpy as jnp
from jax.experimental import pallas as pl


def kernel(feat, edge_index, edge_attr, ada_e_c, ada_e_t, ada_e_d, W_fc, W_fc0, W_fc1, W_fc2, W_fc_src, W_ada_c, W_ada_t, W_ada_d, a_c, a_t, a_d, attn_l, attn_r, bias):
    raise NotImplementedError("write your pallas kernel here")


# speedup vs baseline: 15.2591x; 15.2591x over previous
import functools

import jax
import jax.numpy as jnp
from jax import lax
from jax.experimental import pallas as pl
from jax.experimental.pallas import tpu as pltpu
from jax.experimental.pallas import tpu_sc as plsc

N = 10000
E = 320000
H = 4
FO = 64
NP = 10240
NTILES = 16
EPT = E // NTILES
KB = 80
NBLK = EPT // KB
SLAB = NP // NTILES


def _leaky(x):
    return jnp.where(x >= 0, x, 0.2 * x)


def _t1_body(feat, wfc, wfc1, kmat, smat, arf, fs_ref, nl_ref, er_ref):
    fd = lax.dot_general(feat[...], wfc[...], (((1,), (1,)), ((), ())),
                         preferred_element_type=jnp.float32)
    lfd = _leaky(fd) * arf[...]
    er_ref[...] = lax.dot_general(lfd, smat[...], (((1,), (0,)), ((), ())),
                                  preferred_element_type=jnp.float32)
    fs = _leaky(lax.dot_general(feat[...], wfc1[...], (((1,), (1,)), ((), ())),
                                preferred_element_type=jnp.float32))
    fs_ref[...] = fs
    nl_ref[...] = lax.dot_general(fs, kmat[...], (((1,), (0,)), ((), ())),
                                  preferred_element_type=jnp.float32)


def _t2_body(ea, ac, at_, ad, cmat, vcs, vts, vds, c1_ref, f_ref):
    c1_ref[...] = lax.dot_general(ea[...], cmat[...], (((1,), (0,)), ((), ())),
                                  preferred_element_type=jnp.float32)
    g = (lax.dot_general(ac[...], vcs[...], (((1,), (0,)), ((), ())),
                         preferred_element_type=jnp.float32)
         + lax.dot_general(at_[...], vts[...], (((1,), (0,)), ((), ())),
                           preferred_element_type=jnp.float32)
         + lax.dot_general(ad[...], vds[...], (((1,), (0,)), ((), ())),
                           preferred_element_type=jnp.float32))
    f_ref[...] = jnp.exp(-g)


def _phase_a_body(src_hbm, dst_hbm, cf_hbm, nler_hbm, ex_hbm, denT_hbm,
             nl_t, er_t, src_b, dst_b, c1_b, f_b, ex_b, den_b,
             slab_b, denT_t, den_sh, sem):
    cid = lax.axis_index("c")
    sid = lax.axis_index("s")
    h0 = 2 * cid
    zero16 = jnp.zeros((16,), jnp.float32)

    pltpu.sync_copy(nler_hbm.at[pl.ds(h0 * N, 2 * N)], nl_t)
    pltpu.sync_copy(nler_hbm.at[pl.ds((4 + h0) * N, 2 * N)], er_t)

    for r in range(KB):
        den_b[r, :] = zero16
    for j in range(SLAB // KB):
        pltpu.sync_copy(den_b, den_sh.at[pl.ds(sid * SLAB + j * KB, KB)])
    plsc.subcore_barrier()

    def block(b, carry):
        base = sid * EPT + b * KB
        pltpu.sync_copy(src_hbm.at[pl.ds(base, KB)], src_b)
        pltpu.sync_copy(dst_hbm.at[pl.ds(base, KB)], dst_b)
        for j in range(2):
            pltpu.sync_copy(cf_hbm.at[pl.ds((h0 + j) * E + base, KB)], c1_b.at[j])
            pltpu.sync_copy(cf_hbm.at[pl.ds((4 + h0 + j) * E + base, KB)], f_b.at[j])
        for g in range(KB // 16):
            sl = pl.ds(g * 16, 16)
            s16 = src_b[sl]
            d16 = dst_b[sl]
            lane = lax.iota(jnp.int32, 16)
            for j in range(2):
                nlv = plsc.load_gather(nl_t, [s16 + j * N])
                erv = plsc.load_gather(er_t, [d16 + j * N])
                ev = (nlv + c1_b[j, sl] + erv) * f_b[j, sl]
                ev = jnp.where(ev >= 0, ev, 0.2 * ev)
                exv = jnp.exp(ev)
                ex_b[j, sl] = exv
                plsc.store_scatter(den_b, [lane + g * 16, jnp.full((16,), j, jnp.int32)], exv)
        for j in range(2):
            pltpu.sync_copy(ex_b.at[j], ex_hbm.at[pl.ds((h0 + j) * E + base, KB)])
        pltpu.sync_copy(den_b, den_sh.at[dst_b], add=True)
        return carry

    lax.fori_loop(0, NBLK, block, 0)
    plsc.subcore_barrier()

    n0 = sid * SLAB
    pltpu.sync_copy(den_sh.at[pl.ds(n0, SLAB)], slab_b)
    lane16 = lax.iota(jnp.int32, 16)
    for j in range(2):
        for g in range(SLAB // 16):
            idx = lane16 + g * 16
            v = plsc.load_gather(slab_b, [idx, jnp.full((16,), j, jnp.int32)])
            denT_t[j, pl.ds(g * 16, 16)] = v
    for j in range(2):
        pltpu.sync_copy(denT_t.at[j], denT_hbm.at[pl.ds((h0 + j) * NP + n0, SLAB)])


def _phase_b_body(src_hbm, dst_hbm, ex_hbm, denT_hbm, fs_hbm, binit_hbm, out_hbm,
             dent, src_b, dst_b, ex_b, g_b, msg_b, sem, acc_sh):
    cid = lax.axis_index("c")
    sid = lax.axis_index("s")
    h0 = 2 * cid
    n0 = sid * SLAB

    pltpu.sync_copy(binit_hbm.at[cid, pl.ds(n0, SLAB)], acc_sh.at[pl.ds(n0, SLAB)])

    pltpu.sync_copy(denT_hbm.at[pl.ds(h0 * NP, 2 * NP)], dent)

    def recip(i, carry):
        sl = pl.ds(i * 16, 16)
        dent[sl] = 1.0 / dent[sl]
        return carry

    lax.fori_loop(0, 2 * NP // 16, recip, 0)
    plsc.subcore_barrier()

    def block(b, carry):
        base = sid * EPT + b * KB
        pltpu.sync_copy(src_hbm.at[pl.ds(base, KB)], src_b)
        pltpu.sync_copy(dst_hbm.at[pl.ds(base, KB)], dst_b)
        for j in range(2):
            pltpu.sync_copy(ex_hbm.at[pl.ds((h0 + j) * E + base, KB)], ex_b.at[j])
        pltpu.async_copy(fs_hbm.at[src_b], g_b, sem).wait()
        for g in range(KB // 16):
            sl = pl.ds(g * 16, 16)
            d16 = dst_b[sl]
            avs = []
            for j in range(2):
                invd = plsc.load_gather(dent, [d16 + j * NP])
                av = ex_b[j, sl] * invd
                avs.append(jnp.where(av < 1e-5, 0.0, av))
            for k in range(16):
                ek = g * 16 + k
                for j in range(2):
                    a_s = avs[j][k]
                    for q in range(4):
                        msg_b[ek, pl.ds(j * 64 + q * 16, 16)] = (
                            g_b[ek, pl.ds(q * 16, 16)] * a_s)
        pltpu.sync_copy(msg_b, acc_sh.at[dst_b], add=True)
        return carry

    lax.fori_loop(0, NBLK, block, 0)
    plsc.subcore_barrier()
    pltpu.sync_copy(acc_sh.at[pl.ds(n0, SLAB)], out_hbm.at[cid, pl.ds(n0, SLAB)])


@functools.lru_cache(maxsize=1)
def _sc_kernels():
    mesh = plsc.VectorSubcoreMesh(core_axis_name="c", subcore_axis_name="s")
    phase_a = pl.kernel(
        _phase_a_body,
        mesh=mesh,
        compiler_params=pltpu.CompilerParams(
            needs_layout_passes=False, use_tc_tiling_on_sc=False),
        out_type=[jax.ShapeDtypeStruct((4 * E,), jnp.float32),
                  jax.ShapeDtypeStruct((4 * NP,), jnp.float32)],
        scratch_types=[
            pltpu.VMEM((2 * N,), jnp.float32),
            pltpu.VMEM((2 * N,), jnp.float32),
            pltpu.VMEM((KB,), jnp.int32),
            pltpu.VMEM((KB,), jnp.int32),
            pltpu.VMEM((2, KB), jnp.float32),
            pltpu.VMEM((2, KB), jnp.float32),
            pltpu.VMEM((2, KB), jnp.float32),
            pltpu.VMEM((KB, 16), jnp.float32),
            pltpu.VMEM((SLAB, 16), jnp.float32),
            pltpu.VMEM((2, SLAB), jnp.float32),
            pltpu.VMEM_SHARED((NP, 16), jnp.float32),
            pltpu.SemaphoreType.DMA,
        ],
    )
    phase_b = pl.kernel(
        _phase_b_body,
        mesh=mesh,
        compiler_params=pltpu.CompilerParams(
            needs_layout_passes=False, use_tc_tiling_on_sc=False),
        out_type=jax.ShapeDtypeStruct((2, NP, 128), jnp.float32),
        scratch_types=[
            pltpu.VMEM((2 * NP,), jnp.float32),
            pltpu.VMEM((KB,), jnp.int32),
            pltpu.VMEM((KB,), jnp.int32),
            pltpu.VMEM((2, KB), jnp.float32),
            pltpu.VMEM((KB, 64), jnp.float32),
            pltpu.VMEM((KB, 128), jnp.float32),
            pltpu.SemaphoreType.DMA,
            pltpu.VMEM_SHARED((NP, 128), jnp.float32),
        ],
    )
    return phase_a, phase_b


def kernel(feat, edge_index, edge_attr, ada_e_c, ada_e_t, ada_e_d, W_fc, W_fc0,
           W_fc1, W_fc2, W_fc_src, W_ada_c, W_ada_t, W_ada_d, a_c, a_t, a_d,
           attn_l, attn_r, bias):
    f32 = jnp.float32
    al = attn_l[0]
    u_l = jnp.einsum("hf,hfk->hk", al, W_fc2.reshape(H, FO, FO))
    Wa = W_fc_src[:, :FO]
    Wb = W_fc_src[:, FO:]
    kmat = (jnp.eye(FO, dtype=f32) + Wa.T) @ u_l.T
    cmat = W_fc0.T @ Wb.T @ u_l.T
    vcs = (a_c[0] * W_ada_c.reshape(H, FO, FO).mean(axis=1)).T
    vts = (a_t[0] * W_ada_t.reshape(H, FO, FO).mean(axis=1)).T
    vds = (a_d[0] * W_ada_d.reshape(H, FO, FO).mean(axis=1)).T
    arf = attn_r[0].reshape(1, H * FO)
    smat = jnp.repeat(jnp.eye(H, dtype=f32), FO, axis=0)

    NB = 2000
    fs, nl, er = pl.pallas_call(
        _t1_body,
        grid=(N // NB,),
        in_specs=[
            pl.BlockSpec((NB, 128), lambda i: (i, 0)),
            pl.BlockSpec((H * FO, 128), lambda i: (0, 0)),
            pl.BlockSpec((FO, 128), lambda i: (0, 0)),
            pl.BlockSpec((FO, H), lambda i: (0, 0)),
            pl.BlockSpec((H * FO, H), lambda i: (0, 0)),
            pl.BlockSpec((1, H * FO), lambda i: (0, 0)),
        ],
        out_specs=[
            pl.BlockSpec((NB, FO), lambda i: (i, 0)),
            pl.BlockSpec((NB, H), lambda i: (i, 0)),
            pl.BlockSpec((NB, H), lambda i: (i, 0)),
        ],
        out_shape=[
            jax.ShapeDtypeStruct((N, FO), f32),
            jax.ShapeDtypeStruct((N, H), f32),
            jax.ShapeDtypeStruct((N, H), f32),
        ],
    )(feat, W_fc, W_fc1, kmat, smat, arf)

    EB = 3200
    c1, fdec = pl.pallas_call(
        _t2_body,
        grid=(E // EB,),
        in_specs=[
            pl.BlockSpec((EB, 16), lambda i: (i, 0)),
            pl.BlockSpec((EB, FO), lambda i: (i, 0)),
            pl.BlockSpec((EB, FO), lambda i: (i, 0)),
            pl.BlockSpec((EB, FO), lambda i: (i, 0)),
            pl.BlockSpec((16, H), lambda i: (0, 0)),
            pl.BlockSpec((FO, H), lambda i: (0, 0)),
            pl.BlockSpec((FO, H), lambda i: (0, 0)),
            pl.BlockSpec((FO, H), lambda i: (0, 0)),
        ],
        out_specs=[
            pl.BlockSpec((EB, H), lambda i: (i, 0)),
            pl.BlockSpec((EB, H), lambda i: (i, 0)),
        ],
        out_shape=[
            jax.ShapeDtypeStruct((E, H), f32),
            jax.ShapeDtypeStruct((E, H), f32),
        ],
    )(edge_attr, ada_e_c, ada_e_t, ada_e_d, cmat, vcs, vts, vds)

    src = edge_index[0]
    dst = edge_index[1]
    nler = jnp.concatenate([nl.T, er.T], axis=0).reshape(-1)
    cf = jnp.concatenate([c1.T, fdec.T], axis=0).reshape(-1)
    binit = jnp.broadcast_to(bias.reshape(2, 1, 128), (2, NP, 128))

    phase_a, phase_b = _sc_kernels()
    ex, denT = phase_a(src, dst, cf, nler)
    out = phase_b(src, dst, ex, denT, fs, binit)

    rst = out.reshape(2, NP, 2, FO)[:, :N].transpose(1, 0, 2, 3).reshape(N, H, FO)
    return rst


# speedup vs baseline: 24.7134x; 1.6196x over previous
import functools

import jax
import jax.numpy as jnp
from jax import lax
from jax.experimental import pallas as pl
from jax.experimental.pallas import tpu as pltpu
from jax.experimental.pallas import tpu_sc as plsc

N = 10000
E = 320000
H = 4
FO = 64
NP = 10240
NTILES = 16
EPT = E // NTILES
SUB = 80
NSUB = 10
CH = SUB * NSUB
NCHUNK = EPT // CH
SLAB = NP // NTILES
NSL = N // NTILES
NFPAD = 34000
EIPAD = 2400000


def _leaky(x):
    return jnp.where(x >= 0, x, 0.2 * x)


def _t1_body(feat, wfc, wfc1, kmat, smat, arf, fs_ref, nler_ref):
    fd = lax.dot_general(feat[...], wfc[...], (((1,), (1,)), ((), ())),
                         preferred_element_type=jnp.float32)
    lfd = _leaky(fd) * arf[...]
    fs = _leaky(lax.dot_general(feat[...], wfc1[...], (((1,), (1,)), ((), ())),
                                preferred_element_type=jnp.float32))
    fs_ref[...] = fs
    nler_ref[0:4, :] = lax.dot_general(kmat[...], fs, (((0,), (1,)), ((), ())),
                                       preferred_element_type=jnp.float32)
    nler_ref[4:8, :] = lax.dot_general(smat[...], lfd, (((0,), (1,)), ((), ())),
                                       preferred_element_type=jnp.float32)


def _t2_body(ea, ac, at_, ad, cmat, vcs, vts, vds, cf_ref):
    cf_ref[0:4, :] = lax.dot_general(cmat[...], ea[...], (((0,), (1,)), ((), ())),
                                     preferred_element_type=jnp.float32)
    g = (lax.dot_general(vcs[...], ac[...], (((0,), (1,)), ((), ())),
                         preferred_element_type=jnp.float32)
         + lax.dot_general(vts[...], at_[...], (((0,), (1,)), ((), ())),
                           preferred_element_type=jnp.float32)
         + lax.dot_general(vds[...], ad[...], (((0,), (1,)), ((), ())),
                           preferred_element_type=jnp.float32))
    cf_ref[4:8, :] = jnp.exp(-g)


def _phase_a_body(ei_hbm, cf_hbm, nler_hbm, ex_hbm, denT_hbm,
                  nl_t, er_t, src_f, dst_f, dst_s, c1_b, f_b, ex_b, den_b,
                  slab_b, denT_t, den_sh, sem):
    cid = lax.axis_index("c")
    sid = lax.axis_index("s")
    h0 = 2 * cid
    zero16 = jnp.zeros((16,), jnp.float32)
    lane = lax.iota(jnp.int32, 16)

    pltpu.sync_copy(nler_hbm.at[pl.ds(h0 * N, 2 * N)], nl_t)
    pltpu.sync_copy(nler_hbm.at[pl.ds((4 + h0) * N, 2 * N)], er_t)

    for r in range(CH):
        den_b[r, :] = zero16
    pltpu.sync_copy(den_b.at[pl.ds(0, SLAB)], den_sh.at[pl.ds(sid * SLAB, SLAB)])
    plsc.subcore_barrier()

    def chunk(c, carry):
        base = sid * EPT + c * CH
        cps = [
            pltpu.async_copy(ei_hbm.at[pl.ds(base, CH)], src_f, sem),
            pltpu.async_copy(ei_hbm.at[pl.ds(E + base, CH)], dst_f, sem),
        ]
        for j in range(2):
            cps.append(pltpu.async_copy(
                cf_hbm.at[pl.ds((h0 + j) * E + base, CH)], c1_b.at[j], sem))
            cps.append(pltpu.async_copy(
                cf_hbm.at[pl.ds((4 + h0 + j) * E + base, CH)], f_b.at[j], sem))
        for cp in cps:
            cp.wait()
        for g in range(CH // 16):
            sl = pl.ds(g * 16, 16)
            s16 = src_f[sl]
            d16 = dst_f[sl]
            for j in range(2):
                nlv = plsc.load_gather(nl_t, [s16 + j * N])
                erv = plsc.load_gather(er_t, [d16 + j * N])
                ev = (nlv + c1_b[j, sl] + erv) * f_b[j, sl]
                ev = jnp.where(ev >= 0, ev, 0.2 * ev)
                exv = jnp.exp(ev)
                ex_b[j, sl] = exv
                plsc.store_scatter(
                    den_b, [lane + g * 16, jnp.full((16,), j, jnp.int32)], exv)
        for j in range(2):
            pltpu.sync_copy(ex_b.at[j], ex_hbm.at[pl.ds((h0 + j) * E + base, CH)])
        for i in range(NSUB):
            pltpu.sync_copy(ei_hbm.at[pl.ds(E + base + i * SUB, SUB)], dst_s)
            pltpu.sync_copy(den_b.at[pl.ds(i * SUB, SUB)], den_sh.at[dst_s],
                            add=True)
        return carry

    lax.fori_loop(0, NCHUNK, chunk, 0)
    plsc.subcore_barrier()

    n0 = sid * SLAB
    pltpu.sync_copy(den_sh.at[pl.ds(n0, SLAB)], slab_b)
    for j in range(2):
        for g in range(SLAB // 16):
            idx = lane + g * 16
            v = plsc.load_gather(slab_b, [idx, jnp.full((16,), j, jnp.int32)])
            denT_t[j, pl.ds(g * 16, 16)] = v
    for j in range(2):
        pltpu.sync_copy(denT_t.at[j], denT_hbm.at[pl.ds((h0 + j) * NP + n0, SLAB)])


def _phase_b_body(ei_hbm, ex_hbm, denT_hbm, fs_hbm, bias_hbm, out_hbm,
                  dent, src_s, dst_f, dst_s, ex_b, g0, msg0, bias_v,
                  sem, s0, s1, acc_sh):
    cid = lax.axis_index("c")
    sid = lax.axis_index("s")
    h0 = 2 * cid
    n0 = sid * SLAB

    pltpu.sync_copy(bias_hbm.at[pl.ds(cid * 128, 128)], bias_v)
    brs = [bias_v[pl.ds(q * 16, 16)] for q in range(8)]
    for r in range(16):
        for q in range(8):
            msg0[r, pl.ds(q * 16, 16)] = brs[q]
    cps = []
    for i in range(SLAB // 16):
        cps.append(pltpu.async_copy(msg0.at[pl.ds(0, 16)],
                                    acc_sh.at[pl.ds(n0 + i * 16, 16)], sem))
        if len(cps) >= 8:
            for cp in cps:
                cp.wait()
            cps = []
    for cp in cps:
        cp.wait()

    pltpu.sync_copy(denT_hbm.at[pl.ds(h0 * NP, 2 * NP)], dent)

    def recip(i, carry):
        sl = pl.ds(i * 16, 16)
        dent[sl] = 1.0 / dent[sl]
        return carry

    lax.fori_loop(0, 2 * NP // 16, recip, 0)
    plsc.subcore_barrier()

    def compute_sub(i, gbuf, mbuf):
        o = i * SUB
        for g in range(SUB // 16):
            sl = pl.ds(o + g * 16, 16)
            d16 = dst_f[sl]
            avs = []
            for j in range(2):
                invd = plsc.load_gather(dent, [d16 + j * NP])
                av = ex_b[j, sl] * invd
                avs.append(jnp.where(av < 1e-5, 0.0, av))
            for k in range(16):
                ek = g * 16 + k
                for j in range(2):
                    a_s = avs[j][k]
                    for q in range(4):
                        mbuf[ek, pl.ds(j * 64 + q * 16, 16)] = (
                            gbuf[ek, pl.ds(q * 16, 16)] * a_s)

    def chunk(c, carry):
        base = sid * EPT + c * CH
        cps = [
            pltpu.async_copy(ei_hbm.at[pl.ds(E + base, CH)], dst_f, sem),
            pltpu.async_copy(ex_hbm.at[pl.ds(h0 * E + base, CH)], ex_b.at[0], sem),
            pltpu.async_copy(ex_hbm.at[pl.ds((h0 + 1) * E + base, CH)], ex_b.at[1], sem),
        ]
        for cp in cps:
            cp.wait()

        def sub(i, carry2):
            pltpu.async_copy(
                ei_hbm.at[pl.ds(base + i * SUB, SUB)], src_s, s0).wait()
            pltpu.async_copy(
                ei_hbm.at[pl.ds(E + base + i * SUB, SUB)], dst_s, s1).wait()
            pltpu.async_copy(fs_hbm.at[src_s], g0, s0).wait()
            compute_sub(i, g0, msg0)
            pltpu.async_copy(msg0, acc_sh.at[dst_s], s1, add=True).wait()
            return carry2

        lax.fori_loop(0, NSUB, sub, 0)
        return carry

    lax.fori_loop(0, NCHUNK, chunk, 0)
    plsc.subcore_barrier()
    pltpu.sync_copy(acc_sh.at[pl.ds(n0, SLAB)], out_hbm.at[cid, pl.ds(n0, SLAB)])


@functools.lru_cache(maxsize=1)
def _sc_kernels():
    mesh = plsc.VectorSubcoreMesh(core_axis_name="c", subcore_axis_name="s")
    cparams = pltpu.CompilerParams(
        needs_layout_passes=False, use_tc_tiling_on_sc=False)
    phase_a = pl.kernel(
        _phase_a_body,
        mesh=mesh,
        compiler_params=cparams,
        out_type=[jax.ShapeDtypeStruct((4 * E,), jnp.float32),
                  jax.ShapeDtypeStruct((4 * NP,), jnp.float32)],
        scratch_types=[
            pltpu.VMEM((2 * N,), jnp.float32),
            pltpu.VMEM((2 * N,), jnp.float32),
            pltpu.VMEM((CH,), jnp.int32),
            pltpu.VMEM((CH,), jnp.int32),
            pltpu.VMEM((SUB,), jnp.int32),
            pltpu.VMEM((2, CH), jnp.float32),
            pltpu.VMEM((2, CH), jnp.float32),
            pltpu.VMEM((2, CH), jnp.float32),
            pltpu.VMEM((CH, 16), jnp.float32),
            pltpu.VMEM((SLAB, 16), jnp.float32),
            pltpu.VMEM((2, SLAB), jnp.float32),
            pltpu.VMEM_SHARED((NP, 16), jnp.float32),
            pltpu.SemaphoreType.DMA,
        ],
    )
    phase_b = pl.kernel(
        _phase_b_body,
        mesh=mesh,
        compiler_params=cparams,
        out_type=jax.ShapeDtypeStruct((2, NP, 128), jnp.float32),
        scratch_types=[
            pltpu.VMEM((2 * NP,), jnp.float32),
            pltpu.VMEM((SUB,), jnp.int32),
            pltpu.VMEM((CH,), jnp.int32),
            pltpu.VMEM((SUB,), jnp.int32),
            pltpu.VMEM((2, CH), jnp.float32),
            pltpu.VMEM((SUB, 64), jnp.float32),
            pltpu.VMEM((SUB, 128), jnp.float32),
            pltpu.VMEM((128,), jnp.float32),
            pltpu.SemaphoreType.DMA,
            pltpu.SemaphoreType.DMA,
            pltpu.SemaphoreType.DMA,
            pltpu.VMEM_SHARED((NP, 128), jnp.float32),
        ],
    )
    return phase_a, phase_b


def kernel(feat, edge_index, edge_attr, ada_e_c, ada_e_t, ada_e_d, W_fc, W_fc0,
           W_fc1, W_fc2, W_fc_src, W_ada_c, W_ada_t, W_ada_d, a_c, a_t, a_d,
           attn_l, attn_r, bias):
    f32 = jnp.float32
    al = attn_l[0]
    u_l = jnp.einsum("hf,hfk->hk", al, W_fc2.reshape(H, FO, FO))
    Wa = W_fc_src[:, :FO]
    Wb = W_fc_src[:, FO:]
    kmat = (jnp.eye(FO, dtype=f32) + Wa.T) @ u_l.T
    cmat = W_fc0.T @ Wb.T @ u_l.T
    vcs = (a_c[0] * W_ada_c.reshape(H, FO, FO).mean(axis=1)).T
    vts = (a_t[0] * W_ada_t.reshape(H, FO, FO).mean(axis=1)).T
    vds = (a_d[0] * W_ada_d.reshape(H, FO, FO).mean(axis=1)).T
    arf = attn_r[0].reshape(1, H * FO)
    smat = jnp.repeat(jnp.eye(H, dtype=f32), FO, axis=0)

    fs, nler = pl.pallas_call(
        _t1_body,
        out_shape=[
            jax.ShapeDtypeStruct((N, FO), f32),
            jax.ShapeDtypeStruct((8, N), f32),
        ],
    )(feat, W_fc, W_fc1, kmat, smat, arf)

    EB = 3200
    cf = pl.pallas_call(
        _t2_body,
        grid=(E // EB,),
        in_specs=[
            pl.BlockSpec((EB, 16), lambda i: (i, 0)),
            pl.BlockSpec((EB, FO), lambda i: (i, 0)),
            pl.BlockSpec((EB, FO), lambda i: (i, 0)),
            pl.BlockSpec((EB, FO), lambda i: (i, 0)),
            pl.BlockSpec((16, H), lambda i: (0, 0)),
            pl.BlockSpec((FO, H), lambda i: (0, 0)),
            pl.BlockSpec((FO, H), lambda i: (0, 0)),
            pl.BlockSpec((FO, H), lambda i: (0, 0)),
        ],
        out_specs=pl.BlockSpec((8, EB), lambda i: (0, i)),
        out_shape=jax.ShapeDtypeStruct((8, E), f32),
    )(edge_attr, ada_e_c, ada_e_t, ada_e_d, cmat, vcs, vts, vds)

    phase_a, phase_b = _sc_kernels()
    ei = edge_index.reshape(-1)
    ex, denT = phase_a(ei, cf.reshape(-1), nler.reshape(-1))
    out = phase_b(ei, ex, denT, fs, bias)

    return out.reshape(2, NP, 2, FO)[:, :N].transpose(1, 0, 2, 3).reshape(N, H, FO)


# speedup vs baseline: 28.4591x; 1.1516x over previous
import functools

import jax
import jax.numpy as jnp
from jax import lax
from jax.experimental import pallas as pl
from jax.experimental.pallas import tpu as pltpu
from jax.experimental.pallas import tpu_sc as plsc

N = 10000
E = 320000
H = 4
FO = 64
NP = 10240
NTILES = 16
EPT = E // NTILES
SUB = 80
NSUB = 10
CH = SUB * NSUB
NCHUNK = EPT // CH
SLAB = NP // NTILES
NSL = N // NTILES
NFPAD = 34000
EIPAD = 2400000


def _leaky(x):
    return jnp.where(x >= 0, x, 0.2 * x)


def _t1_body(feat, wfc, wfc1, kmat, smat, arf, fs_ref, nler_ref):
    fd = lax.dot_general(feat[...], wfc[...], (((1,), (1,)), ((), ())),
                         preferred_element_type=jnp.float32)
    lfd = _leaky(fd) * arf[...]
    fs = _leaky(lax.dot_general(feat[...], wfc1[...], (((1,), (1,)), ((), ())),
                                preferred_element_type=jnp.float32))
    fs_ref[...] = fs
    nler_ref[0:4, :] = lax.dot_general(kmat[...], fs, (((0,), (1,)), ((), ())),
                                       preferred_element_type=jnp.float32)
    nler_ref[4:8, :] = lax.dot_general(smat[...], lfd, (((0,), (1,)), ((), ())),
                                       preferred_element_type=jnp.float32)


def _t2_body(ea, ac, at_, ad, cmat, vcs, vts, vds, cf_ref):
    cf_ref[0:4, :] = lax.dot_general(cmat[...], ea[...], (((0,), (1,)), ((), ())),
                                     preferred_element_type=jnp.float32)
    g = (lax.dot_general(vcs[...], ac[...], (((0,), (1,)), ((), ())),
                         preferred_element_type=jnp.float32)
         + lax.dot_general(vts[...], at_[...], (((0,), (1,)), ((), ())),
                           preferred_element_type=jnp.float32)
         + lax.dot_general(vds[...], ad[...], (((0,), (1,)), ((), ())),
                           preferred_element_type=jnp.float32))
    cf_ref[4:8, :] = jnp.exp(-g)


def _phase_a_body(ei_hbm, cf_hbm, nler_hbm, ex_hbm, denT_hbm,
                  nl_t, er_t, src_f, dst_f, dst_s, c1_b, f_b, ex_b, den_b,
                  slab_b, denT_t, den_sh, sem):
    cid = lax.axis_index("c")
    sid = lax.axis_index("s")
    h0 = 2 * cid
    zero16 = jnp.zeros((16,), jnp.float32)
    lane = lax.iota(jnp.int32, 16)

    pltpu.sync_copy(nler_hbm.at[pl.ds(h0 * N, 2 * N)], nl_t)
    pltpu.sync_copy(nler_hbm.at[pl.ds((4 + h0) * N, 2 * N)], er_t)

    for r in range(CH):
        den_b[r, :] = zero16
    pltpu.sync_copy(den_b.at[pl.ds(0, SLAB)], den_sh.at[pl.ds(sid * SLAB, SLAB)])
    plsc.subcore_barrier()

    def chunk(c, carry):
        base = sid * EPT + c * CH
        cps = [
            pltpu.async_copy(ei_hbm.at[pl.ds(base, CH)], src_f, sem),
            pltpu.async_copy(ei_hbm.at[pl.ds(E + base, CH)], dst_f, sem),
        ]
        for j in range(2):
            cps.append(pltpu.async_copy(
                cf_hbm.at[pl.ds((h0 + j) * E + base, CH)], c1_b.at[j], sem))
            cps.append(pltpu.async_copy(
                cf_hbm.at[pl.ds((4 + h0 + j) * E + base, CH)], f_b.at[j], sem))
        for cp in cps:
            cp.wait()
        for g in range(CH // 16):
            sl = pl.ds(g * 16, 16)
            s16 = src_f[sl]
            d16 = dst_f[sl]
            for j in range(2):
                nlv = plsc.load_gather(nl_t, [s16 + j * N])
                erv = plsc.load_gather(er_t, [d16 + j * N])
                ev = (nlv + c1_b[j, sl] + erv) * f_b[j, sl]
                ev = jnp.where(ev >= 0, ev, 0.2 * ev)
                exv = jnp.exp(ev)
                ex_b[j, sl] = exv
                plsc.store_scatter(
                    den_b, [lane + g * 16, jnp.full((16,), j, jnp.int32)], exv)
        for j in range(2):
            pltpu.sync_copy(ex_b.at[j], ex_hbm.at[pl.ds((h0 + j) * E + base, CH)])
        for i in range(NSUB):
            pltpu.sync_copy(ei_hbm.at[pl.ds(E + base + i * SUB, SUB)], dst_s)
            pltpu.sync_copy(den_b.at[pl.ds(i * SUB, SUB)], den_sh.at[dst_s],
                            add=True)
        return carry

    lax.fori_loop(0, NCHUNK, chunk, 0)
    plsc.subcore_barrier()

    n0 = sid * SLAB
    pltpu.sync_copy(den_sh.at[pl.ds(n0, SLAB)], slab_b)
    for j in range(2):
        for g in range(SLAB // 16):
            idx = lane + g * 16
            v = plsc.load_gather(slab_b, [idx, jnp.full((16,), j, jnp.int32)])
            denT_t[j, pl.ds(g * 16, 16)] = v
    for j in range(2):
        pltpu.sync_copy(denT_t.at[j], denT_hbm.at[pl.ds((h0 + j) * NP + n0, SLAB)])


def _phase_b_body(ei_hbm, ex_hbm, denT_hbm, fs_hbm, bias_hbm, out_hbm,
                  dent, src_s, dst_f, dst_s, ex_b, g0, msg0, bias_v,
                  sem, s0, s1, acc_sh):
    cid = lax.axis_index("c")
    sid = lax.axis_index("s")
    h0 = 2 * cid
    n0 = sid * SLAB

    pltpu.sync_copy(bias_hbm.at[pl.ds(cid * 128, 128)], bias_v)
    brs = [bias_v[pl.ds(q * 16, 16)] for q in range(8)]
    for r in range(16):
        for q in range(8):
            msg0[r, pl.ds(q * 16, 16)] = brs[q]
    cps = []
    for i in range(SLAB // 16):
        cps.append(pltpu.async_copy(msg0.at[pl.ds(0, 16)],
                                    acc_sh.at[pl.ds(n0 + i * 16, 16)], sem))
        if len(cps) >= 8:
            for cp in cps:
                cp.wait()
            cps = []
    for cp in cps:
        cp.wait()

    pltpu.sync_copy(denT_hbm.at[pl.ds(h0 * NP, 2 * NP)], dent)

    def recip(i, carry):
        sl = pl.ds(i * 16, 16)
        dent[sl] = 1.0 / dent[sl]
        return carry

    lax.fori_loop(0, 2 * NP // 16, recip, 0)
    plsc.subcore_barrier()

    def compute_sub(i, gbuf, mbuf):
        o = i * SUB
        for g in range(SUB // 16):
            sl = pl.ds(o + g * 16, 16)
            d16 = dst_f[sl]
            avs = []
            for j in range(2):
                invd = plsc.load_gather(dent, [d16 + j * NP])
                av = ex_b[j, sl] * invd
                avs.append(jnp.where(av < 1e-5, 0.0, av))
            for k in range(16):
                ek = g * 16 + k
                for j in range(2):
                    a_s = avs[j][k]
                    for q in range(4):
                        mbuf[ek, pl.ds(j * 64 + q * 16, 16)] = (
                            gbuf[ek, pl.ds(q * 16, 16)] * a_s)

    def chunk(c, carry):
        base = sid * EPT + c * CH
        cps = [
            pltpu.async_copy(ei_hbm.at[pl.ds(E + base, CH)], dst_f, sem),
            pltpu.async_copy(ex_hbm.at[pl.ds(h0 * E + base, CH)], ex_b.at[0], sem),
            pltpu.async_copy(ex_hbm.at[pl.ds((h0 + 1) * E + base, CH)], ex_b.at[1], sem),
        ]
        for cp in cps:
            cp.wait()

        def sub(i, carry2):
            cp_s = pltpu.async_copy(
                ei_hbm.at[pl.ds(base + i * SUB, SUB)], src_s, s0)
            cp_s.wait()
            gcp = pltpu.async_copy(fs_hbm.at[src_s], g0, s0)

            @pl.when(i > 0)
            def _():
                pltpu.make_async_copy(msg0, acc_sh.at[dst_s], s1).wait()

            pltpu.async_copy(
                ei_hbm.at[pl.ds(E + base + i * SUB, SUB)], dst_s, sem).wait()
            gcp.wait()
            compute_sub(i, g0, msg0)
            pltpu.async_copy(msg0, acc_sh.at[dst_s], s1, add=True)
            return carry2

        lax.fori_loop(0, NSUB, sub, 0)
        pltpu.make_async_copy(msg0, acc_sh.at[dst_s], s1).wait()
        return carry

    lax.fori_loop(0, NCHUNK, chunk, 0)
    plsc.subcore_barrier()
    pltpu.sync_copy(acc_sh.at[pl.ds(n0, SLAB)], out_hbm.at[cid, pl.ds(n0, SLAB)])


@functools.lru_cache(maxsize=1)
def _sc_kernels():
    mesh = plsc.VectorSubcoreMesh(core_axis_name="c", subcore_axis_name="s")
    cparams = pltpu.CompilerParams(
        needs_layout_passes=False, use_tc_tiling_on_sc=False)
    phase_a = pl.kernel(
        _phase_a_body,
        mesh=mesh,
        compiler_params=cparams,
        out_type=[jax.ShapeDtypeStruct((4 * E,), jnp.float32),
                  jax.ShapeDtypeStruct((4 * NP,), jnp.float32)],
        scratch_types=[
            pltpu.VMEM((2 * N,), jnp.float32),
            pltpu.VMEM((2 * N,), jnp.float32),
            pltpu.VMEM((CH,), jnp.int32),
            pltpu.VMEM((CH,), jnp.int32),
            pltpu.VMEM((SUB,), jnp.int32),
            pltpu.VMEM((2, CH), jnp.float32),
            pltpu.VMEM((2, CH), jnp.float32),
            pltpu.VMEM((2, CH), jnp.float32),
            pltpu.VMEM((CH, 16), jnp.float32),
            pltpu.VMEM((SLAB, 16), jnp.float32),
            pltpu.VMEM((2, SLAB), jnp.float32),
            pltpu.VMEM_SHARED((NP, 16), jnp.float32),
            pltpu.SemaphoreType.DMA,
        ],
    )
    phase_b = pl.kernel(
        _phase_b_body,
        mesh=mesh,
        compiler_params=cparams,
        out_type=jax.ShapeDtypeStruct((2, NP, 128), jnp.float32),
        scratch_types=[
            pltpu.VMEM((2 * NP,), jnp.float32),
            pltpu.VMEM((SUB,), jnp.int32),
            pltpu.VMEM((CH,), jnp.int32),
            pltpu.VMEM((SUB,), jnp.int32),
            pltpu.VMEM((2, CH), jnp.float32),
            pltpu.VMEM((SUB, 64), jnp.float32),
            pltpu.VMEM((SUB, 128), jnp.float32),
            pltpu.VMEM((128,), jnp.float32),
            pltpu.SemaphoreType.DMA,
            pltpu.SemaphoreType.DMA,
            pltpu.SemaphoreType.DMA,
            pltpu.VMEM_SHARED((NP, 128), jnp.float32),
        ],
    )
    return phase_a, phase_b


def kernel(feat, edge_index, edge_attr, ada_e_c, ada_e_t, ada_e_d, W_fc, W_fc0,
           W_fc1, W_fc2, W_fc_src, W_ada_c, W_ada_t, W_ada_d, a_c, a_t, a_d,
           attn_l, attn_r, bias):
    f32 = jnp.float32
    al = attn_l[0]
    u_l = jnp.einsum("hf,hfk->hk", al, W_fc2.reshape(H, FO, FO))
    Wa = W_fc_src[:, :FO]
    Wb = W_fc_src[:, FO:]
    kmat = (jnp.eye(FO, dtype=f32) + Wa.T) @ u_l.T
    cmat = W_fc0.T @ Wb.T @ u_l.T
    vcs = (a_c[0] * W_ada_c.reshape(H, FO, FO).mean(axis=1)).T
    vts = (a_t[0] * W_ada_t.reshape(H, FO, FO).mean(axis=1)).T
    vds = (a_d[0] * W_ada_d.reshape(H, FO, FO).mean(axis=1)).T
    arf = attn_r[0].reshape(1, H * FO)
    smat = jnp.repeat(jnp.eye(H, dtype=f32), FO, axis=0)

    fs, nler = pl.pallas_call(
        _t1_body,
        out_shape=[
            jax.ShapeDtypeStruct((N, FO), f32),
            jax.ShapeDtypeStruct((8, N), f32),
        ],
    )(feat, W_fc, W_fc1, kmat, smat, arf)

    EB = 12800
    cf = pl.pallas_call(
        _t2_body,
        grid=(E // EB,),
        in_specs=[
            pl.BlockSpec((EB, 16), lambda i: (i, 0)),
            pl.BlockSpec((EB, FO), lambda i: (i, 0)),
            pl.BlockSpec((EB, FO), lambda i: (i, 0)),
            pl.BlockSpec((EB, FO), lambda i: (i, 0)),
            pl.BlockSpec((16, H), lambda i: (0, 0)),
            pl.BlockSpec((FO, H), lambda i: (0, 0)),
            pl.BlockSpec((FO, H), lambda i: (0, 0)),
            pl.BlockSpec((FO, H), lambda i: (0, 0)),
        ],
        out_specs=pl.BlockSpec((8, EB), lambda i: (0, i)),
        out_shape=jax.ShapeDtypeStruct((8, E), f32),
    )(edge_attr, ada_e_c, ada_e_t, ada_e_d, cmat, vcs, vts, vds)

    phase_a, phase_b = _sc_kernels()
    ei = edge_index.reshape(-1)
    ex, denT = phase_a(ei, cf.reshape(-1), nler.reshape(-1))
    out = phase_b(ei, ex, denT, fs, bias)

    return out.reshape(2, NP, 2, FO)[:, :N].transpose(1, 0, 2, 3).reshape(N, H, FO)
